# Initial kernel scaffold; baseline (speedup 1.0000x reference)
import jax
import jax.numpy as jnp
from jax.experimental import pallas as pl


def kernel(x, edge_index, gen_w_src, gen_w_dst, gen_mlp_w1, gen_bn_gamma, gen_bn_beta, gen_mlp_w2, sage_proj_w, sage_proj_b, sage_lin_l_w, sage_lin_l_b, sage_lin_r_w, fc_w, fc_b):
    raise NotImplementedError("write your pallas kernel here")


# speedup vs baseline: 6.5571x; 6.5571x over previous
import functools

import jax
import jax.numpy as jnp
from jax import lax
from jax.experimental import pallas as pl
from jax.experimental.pallas import tpu as pltpu
from jax.experimental.pallas import tpu_sc as plsc

N = 10000
E = 320000
D = 128
H = 64
FH = (D + H) // 2

NC = 2
NS = 16
NW = NC * NS
BE = 80
RPT = 624
TAIL = N - NS * RPT

_MESH = plsc.VectorSubcoreMesh(core_axis_name="c", subcore_axis_name="s")


@functools.partial(
    pl.kernel,
    out_type=jax.ShapeDtypeStruct((NC, N, 2 * H), jnp.float32),
    mesh=_MESH,
    scratch_types=[
        pltpu.VMEM((BE,), jnp.int32),
        pltpu.VMEM((BE,), jnp.int32),
        pltpu.VMEM((BE, 2 * H), jnp.float32),
        pltpu.VMEM_SHARED((N, 2 * H), jnp.float32),
        pltpu.SemaphoreType.DMA,
    ],
)
def _segsum_p(table, src, dst, zrows, out, src_v, dst_v, rows_v, acc, sem):
  cid = lax.axis_index("c")
  sid = lax.axis_index("s")
  wid = sid * NC + cid
  epw = E // NW
  pltpu.sync_copy(zrows, acc.at[pl.ds(sid * RPT, RPT)])

  @pl.when(sid == NS - 1)
  def _():
    pltpu.sync_copy(zrows.at[pl.ds(0, TAIL)], acc.at[pl.ds(NS * RPT, TAIL)])

  plsc.subcore_barrier()
  base = wid * epw

  def body(c, carry):
    off = base + c * BE
    pltpu.sync_copy(src.at[pl.ds(off, BE)], src_v)
    pltpu.sync_copy(dst.at[pl.ds(off, BE)], dst_v)
    pltpu.async_copy(table.at[src_v], rows_v, sem).wait()
    pltpu.sync_copy(rows_v, acc.at[dst_v], add=True)
    return carry

  lax.fori_loop(0, epw // BE, body, 0)
  plsc.subcore_barrier()
  pltpu.sync_copy(acc.at[pl.ds(sid * RPT, RPT)],
                  out.at[cid, pl.ds(sid * RPT, RPT)])

  @pl.when(sid == NS - 1)
  def _():
    pltpu.sync_copy(acc.at[pl.ds(NS * RPT, TAIL)],
                    out.at[cid, pl.ds(NS * RPT, TAIL)])


@functools.partial(
    pl.kernel,
    out_type=jax.ShapeDtypeStruct((NC, N, FH), jnp.float32),
    mesh=_MESH,
    scratch_types=[
        pltpu.VMEM((BE,), jnp.int32),
        pltpu.VMEM((BE,), jnp.int32),
        pltpu.VMEM((BE, FH), jnp.float32),
        pltpu.VMEM_SHARED((N, FH), jnp.float32),
        pltpu.SemaphoreType.DMA,
    ],
    compiler_params=pltpu.CompilerParams(use_tc_tiling_on_sc=False),
)
def _segsum_xs(tab_a, tab_b, src, dst, zrows, out, src_v, dst_v, rows_v, acc,
               sem):
  cid = lax.axis_index("c")
  sid = lax.axis_index("s")
  eps = E // NS
  pltpu.sync_copy(zrows, acc.at[pl.ds(sid * RPT, RPT)])

  @pl.when(sid == NS - 1)
  def _():
    pltpu.sync_copy(zrows.at[pl.ds(0, TAIL)], acc.at[pl.ds(NS * RPT, TAIL)])

  plsc.subcore_barrier()
  base = sid * eps

  def body(c, carry):
    off = base + c * BE
    pltpu.sync_copy(src.at[pl.ds(off, BE)], src_v)
    pltpu.sync_copy(dst.at[pl.ds(off, BE)], dst_v)

    @pl.when(cid == 0)
    def _():
      pltpu.async_copy(tab_a.at[src_v], rows_v, sem).wait()

    @pl.when(cid == 1)
    def _():
      pltpu.async_copy(tab_b.at[src_v], rows_v, sem).wait()

    pltpu.sync_copy(rows_v, acc.at[dst_v], add=True)
    return carry

  lax.fori_loop(0, eps // BE, body, 0)
  plsc.subcore_barrier()
  pltpu.sync_copy(acc.at[pl.ds(sid * RPT, RPT)],
                  out.at[cid, pl.ds(sid * RPT, RPT)])

  @pl.when(sid == NS - 1)
  def _():
    pltpu.sync_copy(acc.at[pl.ds(NS * RPT, TAIL)],
                    out.at[cid, pl.ds(NS * RPT, TAIL)])


def _pre_body(x_ref, ws_ref, wd_ref, p_ref, hd_ref):
  xv = x_ref[...]
  m = jnp.maximum(
      jnp.dot(xv, ws_ref[...], preferred_element_type=jnp.float32), 0.0) + 1e-7
  e = jnp.exp(m)
  p_ref[...] = jnp.concatenate([e, m * e], axis=1)
  hd_ref[...] = jnp.dot(xv, wd_ref[...], preferred_element_type=jnp.float32)


def _mid_body(s_ref, hd_ref, x_ref, w1_ref, g_ref, b_ref, w2_ref, pw_ref,
              pb_ref, xsa_ref, xsb_ref, hc_ref):
  ssum = s_ref[0] + s_ref[1]
  aggr = ssum[:, H:] / (ssum[:, :H] + 1e-16)
  out = aggr + hd_ref[...]
  h1 = jnp.dot(out, w1_ref[...], preferred_element_type=jnp.float32)
  mean = jnp.mean(h1, axis=0, keepdims=True)
  h1c = h1 - mean
  var = jnp.mean(h1c * h1c, axis=0, keepdims=True)
  hn = h1c * lax.rsqrt(var + 1e-5) * g_ref[...] + b_ref[...]
  hn = jnp.maximum(hn, 0.0)
  gen_out = jnp.dot(hn, w2_ref[...], preferred_element_type=jnp.float32)
  hc = jax.nn.sigmoid(jnp.concatenate([x_ref[...], gen_out], axis=1))
  hc_ref[...] = hc
  xs = jnp.maximum(
      jnp.dot(hc, pw_ref[...], preferred_element_type=jnp.float32)
      + pb_ref[...], 0.0)
  xsa_ref[...] = xs[:, :FH]
  xsb_ref[...] = xs[:, FH:]


def _post_body(a_ref, hc_ref, ll_ref, lb_ref, lr_ref, fw_ref, fb_ref,
               prob_ref, logit_ref):
  aggr2 = jnp.concatenate([a_ref[0], a_ref[1]], axis=1)
  h = (jnp.dot(aggr2, ll_ref[...], preferred_element_type=jnp.float32)
       + lb_ref[...]
       + jnp.dot(hc_ref[...], lr_ref[...], preferred_element_type=jnp.float32))
  h = jax.nn.sigmoid(h)
  logits = jnp.sum(h * fw_ref[...], axis=1, keepdims=True) + fb_ref[...]
  logit_ref[...] = logits
  prob_ref[...] = jax.nn.sigmoid(logits)


def kernel(x, edge_index, gen_w_src, gen_w_dst, gen_mlp_w1, gen_bn_gamma,
           gen_bn_beta, gen_mlp_w2, sage_proj_w, sage_proj_b, sage_lin_l_w,
           sage_lin_l_b, sage_lin_r_w, fc_w, fc_b):
  src = edge_index[0]
  dst = edge_index[1]

  p_tab, h_dst = pl.pallas_call(
      _pre_body,
      out_shape=(jax.ShapeDtypeStruct((N, 2 * H), jnp.float32),
                 jax.ShapeDtypeStruct((N, H), jnp.float32)),
  )(x, gen_w_src, gen_w_dst)

  s_part = _segsum_p(p_tab, src, dst, jnp.zeros((RPT, 2 * H), jnp.float32))

  xs_a, xs_b, hc = pl.pallas_call(
      _mid_body,
      out_shape=(jax.ShapeDtypeStruct((N, FH), jnp.float32),
                 jax.ShapeDtypeStruct((N, FH), jnp.float32),
                 jax.ShapeDtypeStruct((N, D + H), jnp.float32)),
  )(s_part, h_dst, x, gen_mlp_w1, gen_bn_gamma.reshape(1, -1),
    gen_bn_beta.reshape(1, -1), gen_mlp_w2, sage_proj_w,
    sage_proj_b.reshape(1, -1))

  a2_part = _segsum_xs(xs_a, xs_b, src, dst,
                       jnp.zeros((RPT, FH), jnp.float32))

  prob, logits = pl.pallas_call(
      _post_body,
      out_shape=(jax.ShapeDtypeStruct((N, 1), jnp.float32),
                 jax.ShapeDtypeStruct((N, 1), jnp.float32)),
  )(a2_part, hc, sage_lin_l_w, sage_lin_l_b.reshape(1, -1), sage_lin_r_w,
    fc_w.reshape(1, -1), fc_b.reshape(1, 1))

  return prob, logits
